# baseline (speedup 1.0000x reference)
import functools

import jax
import jax.numpy as jnp
from jax import lax
from jax.experimental import pallas as pl
from jax.experimental.pallas import tpu as pltpu
from jax.experimental.pallas import tpu_sc as plsc

F32 = jnp.float32
NC = 2
NS = 16
NW = NC * NS


def _pick_block(n):
    if n <= 4096:
        return n
    for b in (2048, 2000, 1024, 1000, 512, 500, 256, 128, 64, 8):
        if n % b == 0:
            return b
    return n


def _fused_mlp(terms, extras, residual, p, interpret=False):
    n = terms[0][0].shape[0]
    d = p["w2"].shape[1]
    blk = _pick_block(n)
    grid = n // blk
    nt = len(terms)
    ne = len(extras)
    has_res = residual is not None

    def body(*refs):
        xs = refs[:nt]
        ws = refs[nt:2 * nt]
        exs = refs[2 * nt:2 * nt + ne]
        pos = 2 * nt + ne
        res = refs[pos] if has_res else None
        pos += 1 if has_res else 0
        b1r, w2r, b2r, gr, br = refs[pos:pos + 5]
        outr = refs[pos + 5]
        s = jnp.dot(xs[0][...], ws[0][...], preferred_element_type=F32)
        for i in range(1, nt):
            s = s + jnp.dot(xs[i][...], ws[i][...], preferred_element_type=F32)
        s = s + b1r[...]
        for ex in exs:
            s = s + ex[...]
        h = s * jax.nn.sigmoid(s)
        y = jnp.dot(h, w2r[...], preferred_element_type=F32) + b2r[...]
        mu = jnp.mean(y, axis=-1, keepdims=True)
        var = jnp.mean((y - mu) * (y - mu), axis=-1, keepdims=True)
        o = (y - mu) * lax.rsqrt(var + 1e-5) * gr[...] + br[...]
        if has_res:
            o = o + res[...]
        outr[...] = o

    in_specs = []
    args = []
    for x, _ in terms:
        in_specs.append(pl.BlockSpec((blk, x.shape[1]), lambda i: (i, 0)))
        args.append(x)
    for _, w in terms:
        in_specs.append(pl.BlockSpec(w.shape, lambda i: (0, 0)))
        args.append(w)
    for ex in extras:
        in_specs.append(pl.BlockSpec((blk, d), lambda i: (i, 0)))
        args.append(ex)
    if has_res:
        in_specs.append(pl.BlockSpec((blk, d), lambda i: (i, 0)))
        args.append(residual)
    vecs = [p["b1"].reshape(1, -1), p["w2"], p["b2"].reshape(1, -1),
            p["g"].reshape(1, -1), p["b"].reshape(1, -1)]
    for v in vecs:
        in_specs.append(pl.BlockSpec(v.shape, lambda i: (0, 0)))
        args.append(v)

    return pl.pallas_call(
        body,
        grid=(grid,),
        in_specs=in_specs,
        out_specs=pl.BlockSpec((blk, d), lambda i: (i, 0)),
        out_shape=jax.ShapeDtypeStruct((n, d), F32),
        interpret=interpret,
    )(*args)


def _matmul(x, w, interpret=False):
    n, k = x.shape
    d = w.shape[1]
    blk = _pick_block(n)

    def body(xr, wr, outr):
        outr[...] = jnp.dot(xr[...], wr[...], preferred_element_type=F32)

    return pl.pallas_call(
        body,
        grid=(n // blk,),
        in_specs=[pl.BlockSpec((blk, k), lambda i: (i, 0)),
                  pl.BlockSpec((k, d), lambda i: (0, 0))],
        out_specs=pl.BlockSpec((blk, d), lambda i: (i, 0)),
        out_shape=jax.ShapeDtypeStruct((n, d), F32),
        interpret=interpret,
    )(x, w)


_CHUNK = 128


def _sc_gather2(ts, td, src, dst):
    e, d = src.shape[0], ts.shape[1]
    n = e // NW
    c = _CHUNK
    nch = n // c
    npair = nch // 2
    mesh = plsc.VectorSubcoreMesh(core_axis_name="c", subcore_axis_name="s")

    @functools.partial(
        pl.kernel, mesh=mesh, name=f"sc_gather2_{e}",
        out_type=(jax.ShapeDtypeStruct((e, d), F32),
                  jax.ShapeDtypeStruct((e, d), F32)),
        scratch_types=[
            [pltpu.VMEM((c,), jnp.int32)] * 4,
            [pltpu.VMEM((c, d), F32)] * 4,
            [pltpu.SemaphoreType.DMA] * 6,
        ],
    )
    def k(ts_h, td_h, src_h, dst_h, os_h, od_h, ibufs, rbufs, sems):
        isA, idA, isB, idB = ibufs
        rsA, rdA, rsB, rdB = rbufs
        ia, ib, ga, gb, wa, wb = sems
        wid = lax.axis_index("s") * NC + lax.axis_index("c")
        base = wid * n

        def sl(j):
            return pl.ds(base + j * c, c)

        pltpu.async_copy(src_h.at[sl(0)], isA, ia)
        pltpu.async_copy(dst_h.at[sl(0)], idA, ia)

        def bodyf(kk, carry):
            a = 2 * kk
            b = a + 1
            a2 = lax.rem(a + 2, nch)

            @pl.when(kk > 0)
            def _():
                pltpu.make_async_copy(rsA, os_h.at[sl(0)], wa).wait()
                pltpu.make_async_copy(rdA, od_h.at[sl(0)], wa).wait()

            pltpu.make_async_copy(src_h.at[sl(0)], isA, ia).wait()
            pltpu.make_async_copy(dst_h.at[sl(0)], idA, ia).wait()
            pltpu.async_copy(ts_h.at[isA], rsA, ga)
            pltpu.async_copy(td_h.at[idA], rdA, ga)

            @pl.when(kk > 0)
            def _():
                pltpu.make_async_copy(rsB, os_h.at[sl(0)], wb).wait()
                pltpu.make_async_copy(rdB, od_h.at[sl(0)], wb).wait()

            pltpu.async_copy(src_h.at[sl(b)], isB, ib)
            pltpu.async_copy(dst_h.at[sl(b)], idB, ib)

            pltpu.make_async_copy(ts_h.at[isA], rsA, ga).wait()
            pltpu.make_async_copy(td_h.at[idA], rdA, ga).wait()
            pltpu.async_copy(rsA, os_h.at[sl(a)], wa)
            pltpu.async_copy(rdA, od_h.at[sl(a)], wa)
            pltpu.async_copy(src_h.at[sl(a2)], isA, ia)
            pltpu.async_copy(dst_h.at[sl(a2)], idA, ia)

            pltpu.make_async_copy(src_h.at[sl(0)], isB, ib).wait()
            pltpu.make_async_copy(dst_h.at[sl(0)], idB, ib).wait()
            pltpu.async_copy(ts_h.at[isB], rsB, gb)
            pltpu.async_copy(td_h.at[idB], rdB, gb)
            pltpu.make_async_copy(ts_h.at[isB], rsB, gb).wait()
            pltpu.make_async_copy(td_h.at[idB], rdB, gb).wait()
            pltpu.async_copy(rsB, os_h.at[sl(b)], wb)
            pltpu.async_copy(rdB, od_h.at[sl(b)], wb)
            return carry

        lax.fori_loop(0, npair, bodyf, 0)
        pltpu.make_async_copy(rsA, os_h.at[sl(0)], wa).wait()
        pltpu.make_async_copy(rdA, od_h.at[sl(0)], wa).wait()
        pltpu.make_async_copy(rsB, os_h.at[sl(0)], wb).wait()
        pltpu.make_async_copy(rdB, od_h.at[sl(0)], wb).wait()
        pltpu.make_async_copy(src_h.at[sl(0)], isA, ia).wait()
        pltpu.make_async_copy(dst_h.at[sl(0)], idA, ia).wait()

    return k(ts, td, src, dst)


def _sc_scatter_add(y, dst, r):
    e, d = y.shape
    n = e // NW
    c = _CHUNK
    nch = n // c
    npair = nch // 2
    zeros = jnp.zeros((r, d), F32)
    mesh = plsc.VectorSubcoreMesh(core_axis_name="c", subcore_axis_name="s")

    @functools.partial(
        pl.kernel, mesh=mesh, name=f"sc_scatter_{e}_{r}",
        out_type=jax.ShapeDtypeStruct((NC, r, d), F32),
        scratch_types=[
            [pltpu.VMEM((c,), jnp.int32)] * 2,
            [pltpu.VMEM((c, d), F32)] * 2,
            pltpu.VMEM_SHARED((r, d), F32),
            [pltpu.SemaphoreType.DMA] * 4,
        ],
    )
    def k(y_h, dst_h, z_h, out_h, ibufs, rbufs, shared, sems):
        idxA, idxB = ibufs
        rowsA, rowsB = rbufs
        la, lb, sa, sb = sems
        cid = lax.axis_index("c")
        sid = lax.axis_index("s")
        wid = sid * NC + cid
        base = wid * n

        def sl(j):
            return pl.ds(base + j * c, c)

        @pl.when(sid == 0)
        def _():
            pltpu.sync_copy(z_h, shared)

        plsc.subcore_barrier()

        pltpu.async_copy(dst_h.at[sl(0)], idxA, la)
        pltpu.async_copy(y_h.at[sl(0)], rowsA, la)

        def bodyf(kk, carry):
            a = 2 * kk
            b = a + 1
            a2 = lax.rem(a + 2, nch)

            @pl.when(kk > 0)
            def _():
                pltpu.make_async_copy(rowsB, shared.at[idxB], sb).wait()

            pltpu.make_async_copy(dst_h.at[sl(0)], idxA, la).wait()
            pltpu.make_async_copy(y_h.at[sl(0)], rowsA, la).wait()
            pltpu.async_copy(rowsA, shared.at[idxA], sa, add=True)

            pltpu.async_copy(dst_h.at[sl(b)], idxB, lb)
            pltpu.async_copy(y_h.at[sl(b)], rowsB, lb)

            pltpu.make_async_copy(rowsA, shared.at[idxA], sa).wait()
            pltpu.make_async_copy(dst_h.at[sl(0)], idxB, lb).wait()
            pltpu.make_async_copy(y_h.at[sl(0)], rowsB, lb).wait()
            pltpu.async_copy(rowsB, shared.at[idxB], sb, add=True)

            pltpu.async_copy(dst_h.at[sl(a2)], idxA, la)
            pltpu.async_copy(y_h.at[sl(a2)], rowsA, la)
            return carry

        lax.fori_loop(0, npair, bodyf, 0)
        pltpu.make_async_copy(rowsB, shared.at[idxB], sb).wait()
        pltpu.make_async_copy(dst_h.at[sl(0)], idxA, la).wait()
        pltpu.make_async_copy(y_h.at[sl(0)], rowsA, la).wait()
        plsc.subcore_barrier()

        @pl.when(sid == 0)
        def _():
            pltpu.sync_copy(shared, out_h.at[cid])

    return k(y, dst, zeros)


def _pad_rows(x, n):
    if x.shape[0] == n:
        return x
    return jnp.pad(x, ((0, n - x.shape[0]), (0, 0)))


def _pad_idx(ix, n, fill):
    if ix.shape[0] == n:
        return ix
    return jnp.pad(ix, (0, n - ix.shape[0]), constant_values=fill)


def _split_w1(w1, k0, k1):
    return w1[:k0], w1[k0:k0 + k1], w1[k0 + k1:]


def kernel(grid_nfeat, mesh_nfeat, g2m_efeat, mesh_efeat, m2g_efeat,
           g2m_src, g2m_dst, mesh_src, mesh_dst, m2g_src, m2g_dst, params):
    ng = grid_nfeat.shape[0]
    nm = mesh_nfeat.shape[0]
    nmp = ((nm + 7) // 8) * 8
    rm = nmp + 8
    rg = ((ng + 7) // 8) * 8 + 8
    tg_trash = rg - 8

    ew = 2 * _CHUNK * NW

    def pad_e(n_edges):
        return ((n_edges + ew - 1) // ew) * ew

    eg2m = pad_e(g2m_efeat.shape[0])
    em_n = pad_e(mesh_efeat.shape[0])
    em2g = pad_e(m2g_efeat.shape[0])

    g = _fused_mlp([(grid_nfeat, params["grid_embed"]["w1"])], [], None,
                   params["grid_embed"])
    m = _fused_mlp([(_pad_rows(mesh_nfeat, nmp), params["mesh_embed"]["w1"])],
                   [], None, params["mesh_embed"])

    ef = jnp.pad(_pad_rows(g2m_efeat, eg2m), ((0, 0), (0, 4)))
    w1e = jnp.pad(params["g2m_eembed"]["w1"], ((0, 4), (0, 0)))
    e = _fused_mlp([(ef, w1e)], [], None, params["g2m_eembed"])

    wa, wb, wc = _split_w1(params["g2m_edge"]["w1"], 128, 128)
    tsrc = _matmul(g, wb)
    tdst = _matmul(m, wc)
    qs, qd = _sc_gather2(tsrc, tdst,
                         _pad_idx(g2m_src, eg2m, 0), _pad_idx(g2m_dst, eg2m, 0))
    e = _fused_mlp([(e, wa)], [qs, qd], e, params["g2m_edge"])

    scat = _sc_scatter_add(e, _pad_idx(g2m_dst, eg2m, nmp), rm)
    wna, wnb, _ = _split_w1(params["g2m_node"]["w1"], 128, 128)
    m = _fused_mlp([(scat[0, :nmp], wna), (scat[1, :nmp], wna), (m, wnb)],
                   [], m, params["g2m_node"])
    g = _fused_mlp([(g, params["grid_enc"]["w1"])], [], g, params["grid_enc"])

    efm = jnp.pad(_pad_rows(mesh_efeat, em_n), ((0, 0), (0, 4)))
    w1m = jnp.pad(params["mesh_eembed"]["w1"], ((0, 4), (0, 0)))
    em = _fused_mlp([(efm, w1m)], [], None, params["mesh_eembed"])

    msrc = _pad_idx(mesh_src, em_n, 0)
    mdst = _pad_idx(mesh_dst, em_n, 0)
    mdst_sc = _pad_idx(mesh_dst, em_n, nmp)
    for lp in params["proc"]:
        wa, wb, wc = _split_w1(lp["edge"]["w1"], 128, 128)
        tcomb = _matmul(m, jnp.concatenate([wb, wc], axis=1))
        qs, qd = _sc_gather2(tcomb[:, :128], tcomb[:, 128:], msrc, mdst)
        em = _fused_mlp([(em, wa)], [qs, qd], em, lp["edge"])
        scat = _sc_scatter_add(em, mdst_sc, rm)
        wna, wnb, _ = _split_w1(lp["node"]["w1"], 128, 128)
        m = _fused_mlp([(scat[0, :nmp], wna), (scat[1, :nmp], wna), (m, wnb)],
                       [], m, lp["node"])

    efd = jnp.pad(_pad_rows(m2g_efeat, em2g), ((0, 0), (0, 4)))
    w1d = jnp.pad(params["m2g_eembed"]["w1"], ((0, 4), (0, 0)))
    ed = _fused_mlp([(efd, w1d)], [], None, params["m2g_eembed"])

    wa, wb, wc = _split_w1(params["m2g_edge"]["w1"], 128, 128)
    tsrc = _matmul(m, wb)
    tdst = _matmul(g, wc)
    qs, qd = _sc_gather2(tsrc, tdst,
                         _pad_idx(m2g_src, em2g, 0), _pad_idx(m2g_dst, em2g, 0))
    ed = _fused_mlp([(ed, wa)], [qs, qd], ed, params["m2g_edge"])

    scat = _sc_scatter_add(ed, _pad_idx(m2g_dst, em2g, tg_trash), rg)
    wna, wnb, _ = _split_w1(params["m2g_node"]["w1"], 128, 128)
    g = _fused_mlp([(scat[0, :ng], wna), (scat[1, :ng], wna), (g, wnb)],
                   [], g, params["m2g_node"])

    return _fused_mlp([(g, params["final"]["w1"])], [], None, params["final"])


# speedup vs baseline: 1.4260x; 1.4260x over previous
import functools

import jax
import jax.numpy as jnp
from jax import lax
from jax.experimental import pallas as pl
from jax.experimental.pallas import tpu as pltpu
from jax.experimental.pallas import tpu_sc as plsc

F32 = jnp.float32
NC = 2
NS = 16
NW = NC * NS


def _pick_block(n):
    if n <= 4096:
        return n
    for b in (2048, 2000, 1024, 1000, 512, 500, 256, 128, 64, 8):
        if n % b == 0:
            return b
    return n


def _fused_mlp(terms, extras, residual, p, interpret=False):
    n = terms[0][0].shape[0]
    d = p["w2"].shape[1]
    blk = _pick_block(n)
    grid = n // blk
    nt = len(terms)
    ne = len(extras)
    has_res = residual is not None

    def body(*refs):
        xs = refs[:nt]
        ws = refs[nt:2 * nt]
        exs = refs[2 * nt:2 * nt + ne]
        pos = 2 * nt + ne
        res = refs[pos] if has_res else None
        pos += 1 if has_res else 0
        b1r, w2r, b2r, gr, br = refs[pos:pos + 5]
        outr = refs[pos + 5]
        s = jnp.dot(xs[0][...], ws[0][...], preferred_element_type=F32)
        for i in range(1, nt):
            s = s + jnp.dot(xs[i][...], ws[i][...], preferred_element_type=F32)
        s = s + b1r[...]
        for ex in exs:
            s = s + ex[...]
        h = s * jax.nn.sigmoid(s)
        y = jnp.dot(h, w2r[...], preferred_element_type=F32) + b2r[...]
        mu = jnp.mean(y, axis=-1, keepdims=True)
        var = jnp.mean((y - mu) * (y - mu), axis=-1, keepdims=True)
        o = (y - mu) * lax.rsqrt(var + 1e-5) * gr[...] + br[...]
        if has_res:
            o = o + res[...]
        outr[...] = o

    in_specs = []
    args = []
    for x, _ in terms:
        in_specs.append(pl.BlockSpec((blk, x.shape[1]), lambda i: (i, 0)))
        args.append(x)
    for _, w in terms:
        in_specs.append(pl.BlockSpec(w.shape, lambda i: (0, 0)))
        args.append(w)
    for ex in extras:
        in_specs.append(pl.BlockSpec((blk, d), lambda i: (i, 0)))
        args.append(ex)
    if has_res:
        in_specs.append(pl.BlockSpec((blk, d), lambda i: (i, 0)))
        args.append(residual)
    vecs = [p["b1"].reshape(1, -1), p["w2"], p["b2"].reshape(1, -1),
            p["g"].reshape(1, -1), p["b"].reshape(1, -1)]
    for v in vecs:
        in_specs.append(pl.BlockSpec(v.shape, lambda i: (0, 0)))
        args.append(v)

    return pl.pallas_call(
        body,
        grid=(grid,),
        in_specs=in_specs,
        out_specs=pl.BlockSpec((blk, d), lambda i: (i, 0)),
        out_shape=jax.ShapeDtypeStruct((n, d), F32),
        interpret=interpret,
    )(*args)


def _matmul(x, w, interpret=False):
    n, k = x.shape
    d = w.shape[1]
    blk = _pick_block(n)

    def body(xr, wr, outr):
        outr[...] = jnp.dot(xr[...], wr[...], preferred_element_type=F32)

    return pl.pallas_call(
        body,
        grid=(n // blk,),
        in_specs=[pl.BlockSpec((blk, k), lambda i: (i, 0)),
                  pl.BlockSpec((k, d), lambda i: (0, 0))],
        out_specs=pl.BlockSpec((blk, d), lambda i: (i, 0)),
        out_shape=jax.ShapeDtypeStruct((n, d), F32),
        interpret=interpret,
    )(x, w)


_CHUNK = 128


def _sc_gather2(ts, td, src, dst, c):
    e, d = src.shape[0], ts.shape[1]
    nt_r, nd_r = ts.shape[0], td.shape[0]
    st_s = False
    st_d = False
    n = e // NW
    nch = n // c
    npair = nch // 2
    mesh = plsc.VectorSubcoreMesh(core_axis_name="c", subcore_axis_name="s")

    @functools.partial(
        pl.kernel, mesh=mesh, name=f"sc_gather2_{e}",
        out_type=(jax.ShapeDtypeStruct((e, d), F32),
                  jax.ShapeDtypeStruct((e, d), F32)),
        scratch_types=[
            [pltpu.VMEM((c,), jnp.int32)] * 4,
            [pltpu.VMEM((c, d), F32)] * 4,
            pltpu.VMEM_SHARED((nt_r if st_s else 8, d), F32),
            pltpu.VMEM_SHARED((nd_r if st_d else 8, d), F32),
            [pltpu.SemaphoreType.DMA] * 6,
        ],
    )
    def k(ts_h, td_h, src_h, dst_h, os_h, od_h, ibufs, rbufs, sh_ts, sh_td, sems):
        isA, idA, isB, idB = ibufs
        rsA, rdA, rsB, rdB = rbufs
        ia, ib, ga, gb, wa, wb = sems
        sid = lax.axis_index("s")
        wid = sid * NC + lax.axis_index("c")
        base = wid * n

        def sl(j):
            return pl.ds(base + j * c, c)

        pltpu.async_copy(src_h.at[sl(0)], isA, ia)
        pltpu.async_copy(dst_h.at[sl(0)], idA, ia)

        if st_s:
            @pl.when(sid == 0)
            def _():
                pltpu.sync_copy(ts_h, sh_ts)

        if st_d:
            @pl.when(sid == 1)
            def _():
                pltpu.sync_copy(td_h, sh_td)

        plsc.subcore_barrier()
        t_s = sh_ts if st_s else ts_h
        t_d = sh_td if st_d else td_h

        def bodyf(kk, carry):
            a = 2 * kk
            b = a + 1
            a2 = lax.rem(a + 2, nch)

            @pl.when(kk > 0)
            def _():
                pltpu.make_async_copy(rsA, os_h.at[sl(0)], wa).wait()
                pltpu.make_async_copy(rdA, od_h.at[sl(0)], wa).wait()

            pltpu.make_async_copy(src_h.at[sl(0)], isA, ia).wait()
            pltpu.make_async_copy(dst_h.at[sl(0)], idA, ia).wait()
            pltpu.async_copy(t_s.at[isA], rsA, ga)
            pltpu.async_copy(t_d.at[idA], rdA, ga)

            @pl.when(kk > 0)
            def _():
                pltpu.make_async_copy(rsB, os_h.at[sl(0)], wb).wait()
                pltpu.make_async_copy(rdB, od_h.at[sl(0)], wb).wait()

            pltpu.async_copy(src_h.at[sl(b)], isB, ib)
            pltpu.async_copy(dst_h.at[sl(b)], idB, ib)

            pltpu.make_async_copy(t_s.at[isA], rsA, ga).wait()
            pltpu.make_async_copy(t_d.at[idA], rdA, ga).wait()
            pltpu.async_copy(rsA, os_h.at[sl(a)], wa)
            pltpu.async_copy(rdA, od_h.at[sl(a)], wa)
            pltpu.async_copy(src_h.at[sl(a2)], isA, ia)
            pltpu.async_copy(dst_h.at[sl(a2)], idA, ia)

            pltpu.make_async_copy(src_h.at[sl(0)], isB, ib).wait()
            pltpu.make_async_copy(dst_h.at[sl(0)], idB, ib).wait()
            pltpu.async_copy(t_s.at[isB], rsB, gb)
            pltpu.async_copy(t_d.at[idB], rdB, gb)
            pltpu.make_async_copy(t_s.at[isB], rsB, gb).wait()
            pltpu.make_async_copy(t_d.at[idB], rdB, gb).wait()
            pltpu.async_copy(rsB, os_h.at[sl(b)], wb)
            pltpu.async_copy(rdB, od_h.at[sl(b)], wb)
            return carry

        lax.fori_loop(0, npair, bodyf, 0)
        pltpu.make_async_copy(rsA, os_h.at[sl(0)], wa).wait()
        pltpu.make_async_copy(rdA, od_h.at[sl(0)], wa).wait()
        pltpu.make_async_copy(rsB, os_h.at[sl(0)], wb).wait()
        pltpu.make_async_copy(rdB, od_h.at[sl(0)], wb).wait()
        pltpu.make_async_copy(src_h.at[sl(0)], isA, ia).wait()
        pltpu.make_async_copy(dst_h.at[sl(0)], idA, ia).wait()

    return k(ts, td, src, dst)


def _sc_scatter_add(y, dst, r, c):
    e, d = y.shape
    n = e // NW
    nch = n // c
    npair = nch // 2
    zeros = jnp.zeros((r, d), F32)
    mesh = plsc.VectorSubcoreMesh(core_axis_name="c", subcore_axis_name="s")

    @functools.partial(
        pl.kernel, mesh=mesh, name=f"sc_scatter_{e}_{r}",
        out_type=jax.ShapeDtypeStruct((NC, r, d), F32),
        scratch_types=[
            [pltpu.VMEM((c,), jnp.int32)] * 2,
            [pltpu.VMEM((c, d), F32)] * 2,
            pltpu.VMEM_SHARED((r, d), F32),
            [pltpu.SemaphoreType.DMA] * 4,
        ],
    )
    def k(y_h, dst_h, z_h, out_h, ibufs, rbufs, shared, sems):
        idxA, idxB = ibufs
        rowsA, rowsB = rbufs
        la, lb, sa, sb = sems
        cid = lax.axis_index("c")
        sid = lax.axis_index("s")
        wid = sid * NC + cid
        base = wid * n

        def sl(j):
            return pl.ds(base + j * c, c)

        @pl.when(sid == 0)
        def _():
            pltpu.sync_copy(z_h, shared)

        plsc.subcore_barrier()

        pltpu.async_copy(dst_h.at[sl(0)], idxA, la)
        pltpu.async_copy(y_h.at[sl(0)], rowsA, la)

        def bodyf(kk, carry):
            a = 2 * kk
            b = a + 1
            a2 = lax.rem(a + 2, nch)

            @pl.when(kk > 0)
            def _():
                pltpu.make_async_copy(rowsB, shared.at[idxB], sb).wait()

            pltpu.make_async_copy(dst_h.at[sl(0)], idxA, la).wait()
            pltpu.make_async_copy(y_h.at[sl(0)], rowsA, la).wait()
            pltpu.async_copy(rowsA, shared.at[idxA], sa, add=True)

            pltpu.async_copy(dst_h.at[sl(b)], idxB, lb)
            pltpu.async_copy(y_h.at[sl(b)], rowsB, lb)

            pltpu.make_async_copy(rowsA, shared.at[idxA], sa).wait()
            pltpu.make_async_copy(dst_h.at[sl(0)], idxB, lb).wait()
            pltpu.make_async_copy(y_h.at[sl(0)], rowsB, lb).wait()
            pltpu.async_copy(rowsB, shared.at[idxB], sb, add=True)

            pltpu.async_copy(dst_h.at[sl(a2)], idxA, la)
            pltpu.async_copy(y_h.at[sl(a2)], rowsA, la)
            return carry

        lax.fori_loop(0, npair, bodyf, 0)
        pltpu.make_async_copy(rowsB, shared.at[idxB], sb).wait()
        pltpu.make_async_copy(dst_h.at[sl(0)], idxA, la).wait()
        pltpu.make_async_copy(y_h.at[sl(0)], rowsA, la).wait()
        plsc.subcore_barrier()

        @pl.when(sid == 0)
        def _():
            pltpu.sync_copy(shared, out_h.at[cid])

    return k(y, dst, zeros)


def _pad_rows(x, n):
    if x.shape[0] == n:
        return x
    return jnp.pad(x, ((0, n - x.shape[0]), (0, 0)))


def _pad_idx(ix, n, fill):
    if ix.shape[0] == n:
        return ix
    return jnp.pad(ix, (0, n - ix.shape[0]), constant_values=fill)


def _pad_and_chunk(n_edges):
    e0 = ((n_edges + 8 * NW - 1) // (8 * NW)) * (8 * NW)
    for e in range(e0, e0 + 65536, 8 * NW):
        n = e // NW
        for c in (128, 120, 112, 104, 96, 88, 80, 72, 64, 56, 48, 40, 32):
            if n % c == 0 and (n // c) % 2 == 0:
                return e, c
    raise ValueError(n_edges)


def _split_w1(w1, k0, k1):
    return w1[:k0], w1[k0:k0 + k1], w1[k0 + k1:]


def kernel(grid_nfeat, mesh_nfeat, g2m_efeat, mesh_efeat, m2g_efeat,
           g2m_src, g2m_dst, mesh_src, mesh_dst, m2g_src, m2g_dst, params):
    ng = grid_nfeat.shape[0]
    nm = mesh_nfeat.shape[0]
    nmp = ((nm + 7) // 8) * 8
    rm = nmp + 8
    rg = ((ng + 7) // 8) * 8 + 8
    tg_trash = rg - 8

    eg2m, c_g2m = _pad_and_chunk(g2m_efeat.shape[0])
    em_n, c_m = _pad_and_chunk(mesh_efeat.shape[0])
    em2g, c_m2g = _pad_and_chunk(m2g_efeat.shape[0])

    g = _fused_mlp([(grid_nfeat, params["grid_embed"]["w1"])], [], None,
                   params["grid_embed"])
    m = _fused_mlp([(_pad_rows(mesh_nfeat, nmp), params["mesh_embed"]["w1"])],
                   [], None, params["mesh_embed"])

    ef = _pad_rows(g2m_efeat, eg2m)
    e = _fused_mlp([(ef, params["g2m_eembed"]["w1"])], [], None,
                   params["g2m_eembed"])

    wa, wb, wc = _split_w1(params["g2m_edge"]["w1"], 128, 128)
    tsrc = _matmul(g, wb)
    tdst = _matmul(m, wc)
    qs, qd = _sc_gather2(tsrc, tdst, _pad_idx(g2m_src, eg2m, 0),
                         _pad_idx(g2m_dst, eg2m, 0), c_g2m)
    e = _fused_mlp([(e, wa)], [qs, qd], e, params["g2m_edge"])

    scat = _sc_scatter_add(e, _pad_idx(g2m_dst, eg2m, nmp), rm, c_g2m)
    wna, wnb, _ = _split_w1(params["g2m_node"]["w1"], 128, 128)
    m = _fused_mlp([(scat[0, :nmp], wna), (scat[1, :nmp], wna), (m, wnb)],
                   [], m, params["g2m_node"])
    g = _fused_mlp([(g, params["grid_enc"]["w1"])], [], g, params["grid_enc"])

    em = _fused_mlp([(_pad_rows(mesh_efeat, em_n), params["mesh_eembed"]["w1"])],
                    [], None, params["mesh_eembed"])

    msrc = _pad_idx(mesh_src, em_n, 0)
    mdst = _pad_idx(mesh_dst, em_n, 0)
    mdst_sc = _pad_idx(mesh_dst, em_n, nmp)
    for lp in params["proc"]:
        wa, wb, wc = _split_w1(lp["edge"]["w1"], 128, 128)
        tcomb = _matmul(m, jnp.concatenate([wb, wc], axis=1))
        qs, qd = _sc_gather2(tcomb[:, :128], tcomb[:, 128:], msrc, mdst, c_m)
        em = _fused_mlp([(em, wa)], [qs, qd], em, lp["edge"])
        scat = _sc_scatter_add(em, mdst_sc, rm, c_m)
        wna, wnb, _ = _split_w1(lp["node"]["w1"], 128, 128)
        m = _fused_mlp([(scat[0, :nmp], wna), (scat[1, :nmp], wna), (m, wnb)],
                       [], m, lp["node"])

    ed = _fused_mlp([(_pad_rows(m2g_efeat, em2g), params["m2g_eembed"]["w1"])],
                    [], None, params["m2g_eembed"])

    wa, wb, wc = _split_w1(params["m2g_edge"]["w1"], 128, 128)
    tsrc = _matmul(m, wb)
    tdst = _matmul(g, wc)
    qs, qd = _sc_gather2(tsrc, tdst, _pad_idx(m2g_src, em2g, 0),
                         _pad_idx(m2g_dst, em2g, 0), c_m2g)
    ed = _fused_mlp([(ed, wa)], [qs, qd], ed, params["m2g_edge"])

    scat = _sc_scatter_add(ed, _pad_idx(m2g_dst, em2g, tg_trash), rg, c_m2g)
    wna, wnb, _ = _split_w1(params["m2g_node"]["w1"], 128, 128)
    g = _fused_mlp([(scat[0, :ng], wna), (scat[1, :ng], wna), (g, wnb)],
                   [], g, params["m2g_node"])

    return _fused_mlp([(g, params["final"]["w1"])], [], None, params["final"])


# speedup vs baseline: 1.4885x; 1.0438x over previous
import functools

import jax
import jax.numpy as jnp
from jax import lax
from jax.experimental import pallas as pl
from jax.experimental.pallas import tpu as pltpu
from jax.experimental.pallas import tpu_sc as plsc

F32 = jnp.float32
NC = 2
NS = 16
NW = NC * NS


def _pick_block(n):
    if n <= 4096:
        return n
    for b in (2048, 2000, 1024, 1000, 512, 500, 256, 128, 64, 8):
        if n % b == 0:
            return b
    return n


def _fused_mlp(terms, extras, residual, p, interpret=False):
    n = terms[0][0].shape[0]
    d = p["w2"].shape[1]
    blk = _pick_block(n)
    grid = n // blk
    nt = len(terms)
    ne = len(extras)
    has_res = residual is not None

    def body(*refs):
        xs = refs[:nt]
        ws = refs[nt:2 * nt]
        exs = refs[2 * nt:2 * nt + ne]
        pos = 2 * nt + ne
        res = refs[pos] if has_res else None
        pos += 1 if has_res else 0
        b1r, w2r, b2r, gr, br = refs[pos:pos + 5]
        outr = refs[pos + 5]
        s = jnp.dot(xs[0][...], ws[0][...], preferred_element_type=F32)
        for i in range(1, nt):
            s = s + jnp.dot(xs[i][...], ws[i][...], preferred_element_type=F32)
        s = s + b1r[...]
        for ex in exs:
            s = s + ex[...]
        h = s * jax.nn.sigmoid(s)
        y = jnp.dot(h, w2r[...], preferred_element_type=F32) + b2r[...]
        mu = jnp.mean(y, axis=-1, keepdims=True)
        var = jnp.mean((y - mu) * (y - mu), axis=-1, keepdims=True)
        o = (y - mu) * lax.rsqrt(var + 1e-5) * gr[...] + br[...]
        if has_res:
            o = o + res[...]
        outr[...] = o

    in_specs = []
    args = []
    for x, _ in terms:
        in_specs.append(pl.BlockSpec((blk, x.shape[1]), lambda i: (i, 0)))
        args.append(x)
    for _, w in terms:
        in_specs.append(pl.BlockSpec(w.shape, lambda i: (0, 0)))
        args.append(w)
    for ex in extras:
        in_specs.append(pl.BlockSpec((blk, d), lambda i: (i, 0)))
        args.append(ex)
    if has_res:
        in_specs.append(pl.BlockSpec((blk, d), lambda i: (i, 0)))
        args.append(residual)
    vecs = [p["b1"].reshape(1, -1), p["w2"], p["b2"].reshape(1, -1),
            p["g"].reshape(1, -1), p["b"].reshape(1, -1)]
    for v in vecs:
        in_specs.append(pl.BlockSpec(v.shape, lambda i: (0, 0)))
        args.append(v)

    return pl.pallas_call(
        body,
        grid=(grid,),
        in_specs=in_specs,
        out_specs=pl.BlockSpec((blk, d), lambda i: (i, 0)),
        out_shape=jax.ShapeDtypeStruct((n, d), F32),
        interpret=interpret,
    )(*args)


def _matmul(x, w, interpret=False):
    n, k = x.shape
    d = w.shape[1]
    blk = _pick_block(n)

    def body(xr, wr, outr):
        outr[...] = jnp.dot(xr[...], wr[...], preferred_element_type=F32)

    return pl.pallas_call(
        body,
        grid=(n // blk,),
        in_specs=[pl.BlockSpec((blk, k), lambda i: (i, 0)),
                  pl.BlockSpec((k, d), lambda i: (0, 0))],
        out_specs=pl.BlockSpec((blk, d), lambda i: (i, 0)),
        out_shape=jax.ShapeDtypeStruct((n, d), F32),
        interpret=interpret,
    )(x, w)


_CHUNK = 128


def _sc_gather2(ts, td, src, dst, c):
    e, d = src.shape[0], ts.shape[1]
    nt_r, nd_r = ts.shape[0], td.shape[0]
    st_s = False
    st_d = False
    n = e // NW
    nch = n // c
    npair = nch // 2
    mesh = plsc.VectorSubcoreMesh(core_axis_name="c", subcore_axis_name="s")

    @functools.partial(
        pl.kernel, mesh=mesh, name=f"sc_gather2_{e}",
        out_type=(jax.ShapeDtypeStruct((e, d), F32),
                  jax.ShapeDtypeStruct((e, d), F32)),
        scratch_types=[
            [pltpu.VMEM((c,), jnp.int32)] * 4,
            [pltpu.VMEM((c, d), F32)] * 4,
            pltpu.VMEM_SHARED((nt_r if st_s else 8, d), F32),
            pltpu.VMEM_SHARED((nd_r if st_d else 8, d), F32),
            [pltpu.SemaphoreType.DMA] * 6,
        ],
    )
    def k(ts_h, td_h, src_h, dst_h, os_h, od_h, ibufs, rbufs, sh_ts, sh_td, sems):
        isA, idA, isB, idB = ibufs
        rsA, rdA, rsB, rdB = rbufs
        ia, ib, ga, gb, wa, wb = sems
        sid = lax.axis_index("s")
        wid = sid * NC + lax.axis_index("c")
        base = wid * n

        def sl(j):
            return pl.ds(base + j * c, c)

        pltpu.async_copy(src_h.at[sl(0)], isA, ia)
        pltpu.async_copy(dst_h.at[sl(0)], idA, ia)

        if st_s:
            @pl.when(sid == 0)
            def _():
                pltpu.sync_copy(ts_h, sh_ts)

        if st_d:
            @pl.when(sid == 1)
            def _():
                pltpu.sync_copy(td_h, sh_td)

        plsc.subcore_barrier()
        t_s = sh_ts if st_s else ts_h
        t_d = sh_td if st_d else td_h

        def bodyf(kk, carry):
            a = 2 * kk
            b = a + 1
            a2 = lax.rem(a + 2, nch)

            @pl.when(kk > 0)
            def _():
                pltpu.make_async_copy(rsA, os_h.at[sl(0)], wa).wait()
                pltpu.make_async_copy(rdA, od_h.at[sl(0)], wa).wait()

            pltpu.make_async_copy(src_h.at[sl(0)], isA, ia).wait()
            pltpu.make_async_copy(dst_h.at[sl(0)], idA, ia).wait()
            pltpu.async_copy(t_s.at[isA], rsA, ga)
            pltpu.async_copy(t_d.at[idA], rdA, ga)

            @pl.when(kk > 0)
            def _():
                pltpu.make_async_copy(rsB, os_h.at[sl(0)], wb).wait()
                pltpu.make_async_copy(rdB, od_h.at[sl(0)], wb).wait()

            pltpu.async_copy(src_h.at[sl(b)], isB, ib)
            pltpu.async_copy(dst_h.at[sl(b)], idB, ib)

            pltpu.make_async_copy(t_s.at[isA], rsA, ga).wait()
            pltpu.make_async_copy(t_d.at[idA], rdA, ga).wait()
            pltpu.async_copy(rsA, os_h.at[sl(a)], wa)
            pltpu.async_copy(rdA, od_h.at[sl(a)], wa)
            pltpu.async_copy(src_h.at[sl(a2)], isA, ia)
            pltpu.async_copy(dst_h.at[sl(a2)], idA, ia)

            pltpu.make_async_copy(src_h.at[sl(0)], isB, ib).wait()
            pltpu.make_async_copy(dst_h.at[sl(0)], idB, ib).wait()
            pltpu.async_copy(t_s.at[isB], rsB, gb)
            pltpu.async_copy(t_d.at[idB], rdB, gb)
            pltpu.make_async_copy(t_s.at[isB], rsB, gb).wait()
            pltpu.make_async_copy(t_d.at[idB], rdB, gb).wait()
            pltpu.async_copy(rsB, os_h.at[sl(b)], wb)
            pltpu.async_copy(rdB, od_h.at[sl(b)], wb)
            return carry

        lax.fori_loop(0, npair, bodyf, 0)
        pltpu.make_async_copy(rsA, os_h.at[sl(0)], wa).wait()
        pltpu.make_async_copy(rdA, od_h.at[sl(0)], wa).wait()
        pltpu.make_async_copy(rsB, os_h.at[sl(0)], wb).wait()
        pltpu.make_async_copy(rdB, od_h.at[sl(0)], wb).wait()
        pltpu.make_async_copy(src_h.at[sl(0)], isA, ia).wait()
        pltpu.make_async_copy(dst_h.at[sl(0)], idA, ia).wait()

    return k(ts, td, src, dst)


def _sc_scatter_add(y, dst, r, c):
    e, d = y.shape
    n = e // NW
    nch = n // c
    npair = nch // 2
    tail = nch - 2 * npair
    zeros = jnp.zeros((r, d), F32)
    mesh = plsc.VectorSubcoreMesh(core_axis_name="c", subcore_axis_name="s")

    @functools.partial(
        pl.kernel, mesh=mesh, name=f"sc_scatter_{e}_{r}",
        out_type=jax.ShapeDtypeStruct((NC, r, d), F32),
        scratch_types=[
            [pltpu.VMEM((c,), jnp.int32)] * 2,
            [pltpu.VMEM((c, d), F32)] * 2,
            pltpu.VMEM_SHARED((r, d), F32),
            [pltpu.SemaphoreType.DMA] * 4,
        ],
    )
    def k(y_h, dst_h, z_h, out_h, ibufs, rbufs, shared, sems):
        idxA, idxB = ibufs
        rowsA, rowsB = rbufs
        la, lb, sa, sb = sems
        cid = lax.axis_index("c")
        sid = lax.axis_index("s")
        wid = sid * NC + cid
        base = wid * n

        def sl(j):
            return pl.ds(base + j * c, c)

        @pl.when(sid == 0)
        def _():
            pltpu.sync_copy(z_h, shared)

        plsc.subcore_barrier()

        pltpu.async_copy(dst_h.at[sl(0)], idxA, la)
        pltpu.async_copy(y_h.at[sl(0)], rowsA, la)

        def bodyf(kk, carry):
            a = 2 * kk
            b = a + 1
            a2 = lax.rem(a + 2, nch)

            @pl.when(kk > 0)
            def _():
                pltpu.make_async_copy(rowsB, shared.at[idxB], sb).wait()

            pltpu.make_async_copy(dst_h.at[sl(0)], idxA, la).wait()
            pltpu.make_async_copy(y_h.at[sl(0)], rowsA, la).wait()
            pltpu.async_copy(rowsA, shared.at[idxA], sa, add=True)

            pltpu.async_copy(dst_h.at[sl(b)], idxB, lb)
            pltpu.async_copy(y_h.at[sl(b)], rowsB, lb)

            pltpu.make_async_copy(rowsA, shared.at[idxA], sa).wait()
            pltpu.make_async_copy(dst_h.at[sl(0)], idxB, lb).wait()
            pltpu.make_async_copy(y_h.at[sl(0)], rowsB, lb).wait()
            pltpu.async_copy(rowsB, shared.at[idxB], sb, add=True)

            pltpu.async_copy(dst_h.at[sl(a2)], idxA, la)
            pltpu.async_copy(y_h.at[sl(a2)], rowsA, la)
            return carry

        lax.fori_loop(0, npair, bodyf, 0)
        pltpu.make_async_copy(rowsB, shared.at[idxB], sb).wait()
        pltpu.make_async_copy(dst_h.at[sl(0)], idxA, la).wait()
        pltpu.make_async_copy(y_h.at[sl(0)], rowsA, la).wait()
        if tail:
            pltpu.async_copy(rowsA, shared.at[idxA], sa, add=True)
            pltpu.make_async_copy(rowsA, shared.at[idxA], sa).wait()
        plsc.subcore_barrier()

        @pl.when(sid == 0)
        def _():
            pltpu.sync_copy(shared, out_h.at[cid])

    return k(y, dst, zeros)


def _pad_rows(x, n):
    if x.shape[0] == n:
        return x
    return jnp.pad(x, ((0, n - x.shape[0]), (0, 0)))


def _pad_idx(ix, n, fill):
    if ix.shape[0] == n:
        return ix
    return jnp.pad(ix, (0, n - ix.shape[0]), constant_values=fill)


def _pad_and_chunk(n_edges):
    e0 = ((n_edges + 8 * NW - 1) // (8 * NW)) * (8 * NW)
    for e in range(e0, e0 + 65536, 8 * NW):
        n = e // NW
        cs = next((c for c in (128, 120, 112, 104, 96, 88, 80, 72, 64, 56, 48,
                               40, 32) if n % c == 0), None)
        cg = next((c for c in (48, 40, 32) if n % c == 0), None)
        if cs and cg and (n // cg) % 2 == 0:
            return e, cg, cs
    raise ValueError(n_edges)


def _split_w1(w1, k0, k1):
    return w1[:k0], w1[k0:k0 + k1], w1[k0 + k1:]


def kernel(grid_nfeat, mesh_nfeat, g2m_efeat, mesh_efeat, m2g_efeat,
           g2m_src, g2m_dst, mesh_src, mesh_dst, m2g_src, m2g_dst, params):
    ng = grid_nfeat.shape[0]
    nm = mesh_nfeat.shape[0]
    nmp = ((nm + 7) // 8) * 8
    rm = nmp + 8
    rg = ((ng + 7) // 8) * 8 + 8
    tg_trash = rg - 8

    eg2m, cg_g2m, cs_g2m = _pad_and_chunk(g2m_efeat.shape[0])
    em_n, cg_m, cs_m = _pad_and_chunk(mesh_efeat.shape[0])
    em2g, cg_m2g, cs_m2g = _pad_and_chunk(m2g_efeat.shape[0])

    g = _fused_mlp([(grid_nfeat, params["grid_embed"]["w1"])], [], None,
                   params["grid_embed"])
    m = _fused_mlp([(_pad_rows(mesh_nfeat, nmp), params["mesh_embed"]["w1"])],
                   [], None, params["mesh_embed"])

    ef = _pad_rows(g2m_efeat, eg2m)
    e = _fused_mlp([(ef, params["g2m_eembed"]["w1"])], [], None,
                   params["g2m_eembed"])

    wa, wb, wc = _split_w1(params["g2m_edge"]["w1"], 128, 128)
    tsrc = _matmul(g, wb)
    tdst = _matmul(m, wc)
    qs, qd = _sc_gather2(tsrc, tdst, _pad_idx(g2m_src, eg2m, 0),
                         _pad_idx(g2m_dst, eg2m, 0), cg_g2m)
    e = _fused_mlp([(e, wa)], [qs, qd], e, params["g2m_edge"])

    scat = _sc_scatter_add(e, _pad_idx(g2m_dst, eg2m, nmp), rm, cs_g2m)
    wna, wnb, _ = _split_w1(params["g2m_node"]["w1"], 128, 128)
    m = _fused_mlp([(scat[0, :nmp], wna), (scat[1, :nmp], wna), (m, wnb)],
                   [], m, params["g2m_node"])
    g = _fused_mlp([(g, params["grid_enc"]["w1"])], [], g, params["grid_enc"])

    e_half = (163840, 156160) if em_n == 320000 else (em_n // 2, em_n - em_n // 2)
    h0 = e_half[0]
    ems = [
        _fused_mlp([(mesh_efeat[:h0], params["mesh_eembed"]["w1"])],
                   [], None, params["mesh_eembed"]),
        _fused_mlp([(mesh_efeat[h0:], params["mesh_eembed"]["w1"])],
                   [], None, params["mesh_eembed"]),
    ]
    msrc_h = (mesh_src[:h0], mesh_src[h0:])
    mdst_h = (mesh_dst[:h0], mesh_dst[h0:])
    for lp in params["proc"]:
        wa, wb, wc = _split_w1(lp["edge"]["w1"], 128, 128)
        wna, wnb, _ = _split_w1(lp["node"]["w1"], 128, 128)
        tcomb = _matmul(m, jnp.concatenate([wb, wc], axis=1))
        ts_t, td_t = tcomb[:, :128], tcomb[:, 128:]
        qsd = [_sc_gather2(ts_t, td_t, msrc_h[h], mdst_h[h], 40)
               for h in range(2)]
        terms = [(m, wnb)]
        for h in range(2):
            ems[h] = _fused_mlp([(ems[h], wa)], list(qsd[h]), ems[h],
                                lp["edge"])
            scat = _sc_scatter_add(ems[h], mdst_h[h], rm,
                                   128 if h == 0 else 80)
            terms += [(scat[0, :nmp], wna), (scat[1, :nmp], wna)]
        m = _fused_mlp(terms, [], m, lp["node"])

    ed = _fused_mlp([(_pad_rows(m2g_efeat, em2g), params["m2g_eembed"]["w1"])],
                    [], None, params["m2g_eembed"])

    wa, wb, wc = _split_w1(params["m2g_edge"]["w1"], 128, 128)
    tsrc = _matmul(m, wb)
    tdst = _matmul(g, wc)
    qs, qd = _sc_gather2(tsrc, tdst, _pad_idx(m2g_src, em2g, 0),
                         _pad_idx(m2g_dst, em2g, 0), cg_m2g)
    ed = _fused_mlp([(ed, wa)], [qs, qd], ed, params["m2g_edge"])

    scat = _sc_scatter_add(ed, _pad_idx(m2g_dst, em2g, tg_trash), rg, cs_m2g)
    wna, wnb, _ = _split_w1(params["m2g_node"]["w1"], 128, 128)
    g = _fused_mlp([(scat[0, :ng], wna), (scat[1, :ng], wna), (g, wnb)],
                   [], g, params["m2g_node"])

    return _fused_mlp([(g, params["final"]["w1"])], [], None, params["final"])


# speedup vs baseline: 1.7195x; 1.1552x over previous
import functools

import jax
import jax.numpy as jnp
from jax import lax
from jax.experimental import pallas as pl
from jax.experimental.pallas import tpu as pltpu
from jax.experimental.pallas import tpu_sc as plsc

F32 = jnp.float32
NC = 2
NS = 16
NW = NC * NS


def _pick_block(n):
    if n <= 4096:
        return n
    for b in range(2048, 7, -8):
        if n % b == 0:
            return b
    return n


def _fused_mlp(terms, extras, residual, p, interpret=False):
    n = terms[0][0].shape[0]
    d = p["w2"].shape[1]
    blk = _pick_block(n)
    grid = n // blk
    nt = len(terms)
    ne = len(extras)
    has_res = residual is not None

    def body(*refs):
        xs = refs[:nt]
        ws = refs[nt:2 * nt]
        exs = refs[2 * nt:2 * nt + ne]
        pos = 2 * nt + ne
        res = refs[pos] if has_res else None
        pos += 1 if has_res else 0
        b1r, w2r, b2r, gr, br = refs[pos:pos + 5]
        outr = refs[pos + 5]
        s = jnp.dot(xs[0][...], ws[0][...], preferred_element_type=F32)
        for i in range(1, nt):
            s = s + jnp.dot(xs[i][...], ws[i][...], preferred_element_type=F32)
        s = s + b1r[...]
        for ex in exs:
            v = ex[...]
            if v.dtype == jnp.int32:
                fa = lax.bitcast_convert_type(v & jnp.int32(-65536), F32)
                fb = lax.bitcast_convert_type(v << 16, F32)
                v = jnp.concatenate([fa, fb], axis=-1)
            s = s + v
        h = s * jax.nn.sigmoid(s)
        y = jnp.dot(h, w2r[...], preferred_element_type=F32) + b2r[...]
        mu = jnp.mean(y, axis=-1, keepdims=True)
        var = jnp.mean((y - mu) * (y - mu), axis=-1, keepdims=True)
        o = (y - mu) * lax.rsqrt(var + 1e-5) * gr[...] + br[...]
        if has_res:
            o = o + res[...]
        outr[...] = o

    in_specs = []
    args = []
    for x, _ in terms:
        in_specs.append(pl.BlockSpec((blk, x.shape[1]), lambda i: (i, 0)))
        args.append(x)
    for _, w in terms:
        in_specs.append(pl.BlockSpec(w.shape, lambda i: (0, 0)))
        args.append(w)
    for ex in extras:
        in_specs.append(pl.BlockSpec((blk, ex.shape[1]), lambda i: (i, 0)))
        args.append(ex)
    if has_res:
        in_specs.append(pl.BlockSpec((blk, d), lambda i: (i, 0)))
        args.append(residual)
    vecs = [p["b1"].reshape(1, -1), p["w2"], p["b2"].reshape(1, -1),
            p["g"].reshape(1, -1), p["b"].reshape(1, -1)]
    for v in vecs:
        in_specs.append(pl.BlockSpec(v.shape, lambda i: (0, 0)))
        args.append(v)

    return pl.pallas_call(
        body,
        grid=(grid,),
        in_specs=in_specs,
        out_specs=pl.BlockSpec((blk, d), lambda i: (i, 0)),
        out_shape=jax.ShapeDtypeStruct((n, d), F32),
        interpret=interpret,
    )(*args)


def _matmul_pack(x, w, interpret=False):
    n, k = x.shape
    d = w.shape[1]
    blk = _pick_block(n)

    def body(xr, wr, outr):
        y = jnp.dot(xr[...], wr[...], preferred_element_type=F32)
        parts = []
        for g0 in range(0, d, 128):
            ua = lax.bitcast_convert_type(y[:, g0:g0 + 64], jnp.int32) + 0x8000
            ub = lax.bitcast_convert_type(y[:, g0 + 64:g0 + 128],
                                          jnp.int32) + 0x8000
            parts.append((ua & jnp.int32(-65536))
                         | lax.shift_right_logical(ub, 16))
        outr[...] = parts[0] if len(parts) == 1 else jnp.concatenate(parts, -1)

    return pl.pallas_call(
        body,
        grid=(n // blk,),
        in_specs=[pl.BlockSpec((blk, k), lambda i: (i, 0)),
                  pl.BlockSpec((k, d), lambda i: (0, 0))],
        out_specs=pl.BlockSpec((blk, d // 2), lambda i: (i, 0)),
        out_shape=jax.ShapeDtypeStruct((n, d // 2), jnp.int32),
        interpret=interpret,
    )(x, w)


def _matmul(x, w, interpret=False):
    n, k = x.shape
    d = w.shape[1]
    blk = _pick_block(n)

    def body(xr, wr, outr):
        outr[...] = jnp.dot(xr[...], wr[...], preferred_element_type=F32)

    return pl.pallas_call(
        body,
        grid=(n // blk,),
        in_specs=[pl.BlockSpec((blk, k), lambda i: (i, 0)),
                  pl.BlockSpec((k, d), lambda i: (0, 0))],
        out_specs=pl.BlockSpec((blk, d), lambda i: (i, 0)),
        out_shape=jax.ShapeDtypeStruct((n, d), F32),
        interpret=interpret,
    )(x, w)


_CHUNK = 128


def _sc_gather2(ts, td, src, dst, c):
    e, d = src.shape[0], ts.shape[1]
    dt = ts.dtype
    nt_r, nd_r = ts.shape[0], td.shape[0]
    st_s = False
    st_d = False
    n = e // NW
    nch = n // c
    npair = nch // 2
    mesh = plsc.VectorSubcoreMesh(core_axis_name="c", subcore_axis_name="s")

    @functools.partial(
        pl.kernel, mesh=mesh, name=f"sc_gather2_{e}",
        out_type=(jax.ShapeDtypeStruct((e, d), dt),
                  jax.ShapeDtypeStruct((e, d), dt)),
        scratch_types=[
            [pltpu.VMEM((c,), jnp.int32)] * 4,
            [pltpu.VMEM((c, d), dt)] * 4,
            pltpu.VMEM_SHARED((nt_r if st_s else 8, d), dt),
            pltpu.VMEM_SHARED((nd_r if st_d else 8, d), dt),
            [pltpu.SemaphoreType.DMA] * 6,
        ],
    )
    def k(ts_h, td_h, src_h, dst_h, os_h, od_h, ibufs, rbufs, sh_ts, sh_td, sems):
        isA, idA, isB, idB = ibufs
        rsA, rdA, rsB, rdB = rbufs
        ia, ib, ga, gb, wa, wb = sems
        sid = lax.axis_index("s")
        wid = sid * NC + lax.axis_index("c")
        base = wid * n

        def sl(j):
            return pl.ds(base + j * c, c)

        pltpu.async_copy(src_h.at[sl(0)], isA, ia)
        pltpu.async_copy(dst_h.at[sl(0)], idA, ia)

        if st_s:
            @pl.when(sid == 0)
            def _():
                pltpu.sync_copy(ts_h, sh_ts)

        if st_d:
            @pl.when(sid == 1)
            def _():
                pltpu.sync_copy(td_h, sh_td)

        plsc.subcore_barrier()
        t_s = sh_ts if st_s else ts_h
        t_d = sh_td if st_d else td_h

        def bodyf(kk, carry):
            a = 2 * kk
            b = a + 1
            a2 = lax.rem(a + 2, nch)

            @pl.when(kk > 0)
            def _():
                pltpu.make_async_copy(rsA, os_h.at[sl(0)], wa).wait()
                pltpu.make_async_copy(rdA, od_h.at[sl(0)], wa).wait()

            pltpu.make_async_copy(src_h.at[sl(0)], isA, ia).wait()
            pltpu.make_async_copy(dst_h.at[sl(0)], idA, ia).wait()
            pltpu.async_copy(t_s.at[isA], rsA, ga)
            pltpu.async_copy(t_d.at[idA], rdA, ga)

            @pl.when(kk > 0)
            def _():
                pltpu.make_async_copy(rsB, os_h.at[sl(0)], wb).wait()
                pltpu.make_async_copy(rdB, od_h.at[sl(0)], wb).wait()

            pltpu.async_copy(src_h.at[sl(b)], isB, ib)
            pltpu.async_copy(dst_h.at[sl(b)], idB, ib)

            pltpu.make_async_copy(t_s.at[isA], rsA, ga).wait()
            pltpu.make_async_copy(t_d.at[idA], rdA, ga).wait()
            pltpu.async_copy(rsA, os_h.at[sl(a)], wa)
            pltpu.async_copy(rdA, od_h.at[sl(a)], wa)
            pltpu.async_copy(src_h.at[sl(a2)], isA, ia)
            pltpu.async_copy(dst_h.at[sl(a2)], idA, ia)

            pltpu.make_async_copy(src_h.at[sl(0)], isB, ib).wait()
            pltpu.make_async_copy(dst_h.at[sl(0)], idB, ib).wait()
            pltpu.async_copy(t_s.at[isB], rsB, gb)
            pltpu.async_copy(t_d.at[idB], rdB, gb)
            pltpu.make_async_copy(t_s.at[isB], rsB, gb).wait()
            pltpu.make_async_copy(t_d.at[idB], rdB, gb).wait()
            pltpu.async_copy(rsB, os_h.at[sl(b)], wb)
            pltpu.async_copy(rdB, od_h.at[sl(b)], wb)
            return carry

        lax.fori_loop(0, npair, bodyf, 0)
        pltpu.make_async_copy(rsA, os_h.at[sl(0)], wa).wait()
        pltpu.make_async_copy(rdA, od_h.at[sl(0)], wa).wait()
        pltpu.make_async_copy(rsB, os_h.at[sl(0)], wb).wait()
        pltpu.make_async_copy(rdB, od_h.at[sl(0)], wb).wait()
        pltpu.make_async_copy(src_h.at[sl(0)], isA, ia).wait()
        pltpu.make_async_copy(dst_h.at[sl(0)], idA, ia).wait()

    return k(ts, td, src, dst)


def _sc_scatter_add(y, dst, r, c):
    e, d = y.shape
    n = e // NW
    nch = n // c
    npair = nch // 2
    tail = nch - 2 * npair
    zeros = jnp.zeros((r, d), F32)
    mesh = plsc.VectorSubcoreMesh(core_axis_name="c", subcore_axis_name="s")

    @functools.partial(
        pl.kernel, mesh=mesh, name=f"sc_scatter_{e}_{r}",
        out_type=jax.ShapeDtypeStruct((NC, r, d), F32),
        scratch_types=[
            [pltpu.VMEM((c,), jnp.int32)] * 2,
            [pltpu.VMEM((c, d), F32)] * 2,
            pltpu.VMEM_SHARED((r, d), F32),
            [pltpu.SemaphoreType.DMA] * 4,
        ],
    )
    def k(y_h, dst_h, z_h, out_h, ibufs, rbufs, shared, sems):
        idxA, idxB = ibufs
        rowsA, rowsB = rbufs
        la, lb, sa, sb = sems
        cid = lax.axis_index("c")
        sid = lax.axis_index("s")
        wid = sid * NC + cid
        base = wid * n

        def sl(j):
            return pl.ds(base + j * c, c)

        @pl.when(sid == 0)
        def _():
            pltpu.sync_copy(z_h, shared)

        plsc.subcore_barrier()

        pltpu.async_copy(dst_h.at[sl(0)], idxA, la)
        pltpu.async_copy(y_h.at[sl(0)], rowsA, la)

        def bodyf(kk, carry):
            a = 2 * kk
            b = a + 1
            a2 = lax.rem(a + 2, nch)

            @pl.when(kk > 0)
            def _():
                pltpu.make_async_copy(rowsB, shared.at[idxB], sb).wait()

            pltpu.make_async_copy(dst_h.at[sl(0)], idxA, la).wait()
            pltpu.make_async_copy(y_h.at[sl(0)], rowsA, la).wait()
            pltpu.async_copy(rowsA, shared.at[idxA], sa, add=True)

            pltpu.async_copy(dst_h.at[sl(b)], idxB, lb)
            pltpu.async_copy(y_h.at[sl(b)], rowsB, lb)

            pltpu.make_async_copy(rowsA, shared.at[idxA], sa).wait()
            pltpu.make_async_copy(dst_h.at[sl(0)], idxB, lb).wait()
            pltpu.make_async_copy(y_h.at[sl(0)], rowsB, lb).wait()
            pltpu.async_copy(rowsB, shared.at[idxB], sb, add=True)

            pltpu.async_copy(dst_h.at[sl(a2)], idxA, la)
            pltpu.async_copy(y_h.at[sl(a2)], rowsA, la)
            return carry

        lax.fori_loop(0, npair, bodyf, 0)
        pltpu.make_async_copy(rowsB, shared.at[idxB], sb).wait()
        pltpu.make_async_copy(dst_h.at[sl(0)], idxA, la).wait()
        pltpu.make_async_copy(y_h.at[sl(0)], rowsA, la).wait()
        if tail:
            pltpu.async_copy(rowsA, shared.at[idxA], sa, add=True)
            pltpu.make_async_copy(rowsA, shared.at[idxA], sa).wait()
        plsc.subcore_barrier()

        @pl.when(sid == 0)
        def _():
            pltpu.sync_copy(shared, out_h.at[cid])

    return k(y, dst, zeros)


def _pad_rows(x, n):
    if x.shape[0] == n:
        return x
    return jnp.pad(x, ((0, n - x.shape[0]), (0, 0)))


def _pad_idx(ix, n, fill):
    if ix.shape[0] == n:
        return ix
    return jnp.pad(ix, (0, n - ix.shape[0]), constant_values=fill)


def _pad_and_chunk(n_edges):
    e0 = ((n_edges + 8 * NW - 1) // (8 * NW)) * (8 * NW)
    for e in range(e0, e0 + 65536, 8 * NW):
        n = e // NW
        cs = next((c for c in (128, 120, 112, 104, 96, 88, 80, 72, 64, 56, 48,
                               40, 32) if n % c == 0), None)
        cg = next((c for c in (48, 40, 32) if n % c == 0), None)
        if cs and cg and (n // cg) % 2 == 0:
            return e, cg, cs
    raise ValueError(n_edges)


def _split_w1(w1, k0, k1):
    return w1[:k0], w1[k0:k0 + k1], w1[k0 + k1:]


def kernel(grid_nfeat, mesh_nfeat, g2m_efeat, mesh_efeat, m2g_efeat,
           g2m_src, g2m_dst, mesh_src, mesh_dst, m2g_src, m2g_dst, params):
    ng = grid_nfeat.shape[0]
    nm = mesh_nfeat.shape[0]
    nmp = ((nm + 7) // 8) * 8
    rm = nmp + 8
    rg = ((ng + 7) // 8) * 8 + 8
    tg_trash = rg - 8

    eg2m, cg_g2m, cs_g2m = _pad_and_chunk(g2m_efeat.shape[0])
    em_n, cg_m, cs_m = _pad_and_chunk(mesh_efeat.shape[0])
    em2g, cg_m2g, cs_m2g = _pad_and_chunk(m2g_efeat.shape[0])

    g = _fused_mlp([(grid_nfeat, params["grid_embed"]["w1"])], [], None,
                   params["grid_embed"])
    m = _fused_mlp([(_pad_rows(mesh_nfeat, nmp), params["mesh_embed"]["w1"])],
                   [], None, params["mesh_embed"])

    ef = _pad_rows(g2m_efeat, eg2m)
    e = _fused_mlp([(ef, params["g2m_eembed"]["w1"])], [], None,
                   params["g2m_eembed"])

    wa, wb, wc = _split_w1(params["g2m_edge"]["w1"], 128, 128)
    tsrc = _matmul(g, wb)
    tdst = _matmul(m, wc)
    qs, qd = _sc_gather2(tsrc, tdst, _pad_idx(g2m_src, eg2m, 0),
                         _pad_idx(g2m_dst, eg2m, 0), cg_g2m)
    e = _fused_mlp([(e, wa)], [qs, qd], e, params["g2m_edge"])

    scat = _sc_scatter_add(e, _pad_idx(g2m_dst, eg2m, nmp), rm, cs_g2m)
    wna, wnb, _ = _split_w1(params["g2m_node"]["w1"], 128, 128)
    m = _fused_mlp([(scat[0, :nmp], wna), (scat[1, :nmp], wna), (m, wnb)],
                   [], m, params["g2m_node"])
    g = _fused_mlp([(g, params["grid_enc"]["w1"])], [], g, params["grid_enc"])

    e_half = (163840, 156160) if em_n == 320000 else (em_n // 2, em_n - em_n // 2)
    h0 = e_half[0]
    ems = [
        _fused_mlp([(mesh_efeat[:h0], params["mesh_eembed"]["w1"])],
                   [], None, params["mesh_eembed"]),
        _fused_mlp([(mesh_efeat[h0:], params["mesh_eembed"]["w1"])],
                   [], None, params["mesh_eembed"]),
    ]
    msrc_h = (mesh_src[:h0], mesh_src[h0:])
    mdst_h = (mesh_dst[:h0], mesh_dst[h0:])
    for lp in params["proc"]:
        wa, wb, wc = _split_w1(lp["edge"]["w1"], 128, 128)
        wna, wnb, _ = _split_w1(lp["node"]["w1"], 128, 128)
        tcomb = _matmul(m, jnp.concatenate([wb, wc], axis=1))
        ts_t, td_t = tcomb[:, :128], tcomb[:, 128:]
        qsd = [_sc_gather2(ts_t, td_t, msrc_h[h], mdst_h[h], 40)
               for h in range(2)]
        terms = [(m, wnb)]
        for h in range(2):
            ems[h] = _fused_mlp([(ems[h], wa)], list(qsd[h]), ems[h],
                                lp["edge"])
            scat = _sc_scatter_add(ems[h], mdst_h[h], rm,
                                   128 if h == 0 else 80)
            terms += [(scat[0, :nmp], wna), (scat[1, :nmp], wna)]
        m = _fused_mlp(terms, [], m, lp["node"])

    ed = _fused_mlp([(_pad_rows(m2g_efeat, em2g), params["m2g_eembed"]["w1"])],
                    [], None, params["m2g_eembed"])

    wa, wb, wc = _split_w1(params["m2g_edge"]["w1"], 128, 128)
    tsrc = _matmul(m, wb)
    tdst = _matmul(g, wc)
    qs, qd = _sc_gather2(tsrc, tdst, _pad_idx(m2g_src, em2g, 0),
                         _pad_idx(m2g_dst, em2g, 0), cg_m2g)
    ed = _fused_mlp([(ed, wa)], [qs, qd], ed, params["m2g_edge"])

    scat = _sc_scatter_add(ed, _pad_idx(m2g_dst, em2g, tg_trash), rg, cs_m2g)
    wna, wnb, _ = _split_w1(params["m2g_node"]["w1"], 128, 128)
    g = _fused_mlp([(scat[0, :ng], wna), (scat[1, :ng], wna), (g, wnb)],
                   [], g, params["m2g_node"])

    return _fused_mlp([(g, params["final"]["w1"])], [], None, params["final"])
